# baseline (speedup 1.0000x reference)
import functools

import jax
import jax.numpy as jnp
from jax import lax
from jax.experimental import pallas as pl
from jax.experimental.pallas import tpu as pltpu
from jax.experimental.pallas import tpu_sc as plsc

B = 16384
H = 50
D = 32
DP = 64
NW = 32
CB = B // NW
NPAIR = CB // 2
TR = 250000
HSS = 104
HCS = 52
PXS = 32


def _sqrt16(x):
    xs = jnp.maximum(x, 1e-12)
    i = lax.bitcast_convert_type(xs, jnp.int32)
    y = lax.bitcast_convert_type(jnp.int32(0x5F3759DF) - (i >> 1), jnp.float32)
    y = y * (1.5 - 0.5 * xs * y * y)
    y = y * (1.5 - 0.5 * xs * y * y)
    return xs * y


def _sc_body(hs_hbm, hc_hbm, px_hbm, dist_hbm,
             up_hbm, pp_hbm, gi_hbm, gs_hbm, out_hbm,
             hs_v, hc_v, px_v, dpf_v,
             bA, bB, uA, uB, pA, pB, sA, sB, w_v, out_v, semA, semB):
    wid = lax.axis_index("c") * 16 + lax.axis_index("s")
    base = wid * CB

    pltpu.sync_copy(hs_hbm.at[pl.ds(wid * NPAIR * HSS, NPAIR * HSS)], hs_v)
    pltpu.sync_copy(hc_hbm.at[pl.ds(wid * NPAIR * HCS, NPAIR * HCS)], hc_v)
    pltpu.sync_copy(px_hbm.at[pl.ds(wid * NPAIR * PXS, NPAIR * PXS)], px_v)
    pltpu.sync_copy(dist_hbm.at[pl.ds(base * DP, CB * DP)], dpf_v)

    def _sqrt_step(i, c):
        sl = pl.ds(i * 16, 16)
        dpf_v[sl] = _sqrt16(dpf_v[sl])
        return c
    lax.fori_loop(0, CB * DP // 16, _sqrt_step, 0)

    def start(p, b_, u_, pp_, s_, sem):
        pltpu.async_copy(gi_hbm.at[hs_v.at[pl.ds(p * HSS, 2 * H)]], b_, sem)
        pltpu.async_copy(up_hbm.at[px_v.at[pl.ds(p * PXS + 8, 2)]], u_, sem)
        pltpu.async_copy(pp_hbm.at[px_v.at[pl.ds(p * PXS + 16, 2)]], pp_, sem)
        pltpu.async_copy(gs_hbm.at[px_v.at[pl.ds(p * PXS + 16, 2)]], s_, sem)

    def wait(p, b_, u_, pp_, s_, sem):
        pltpu.make_async_copy(
            gi_hbm.at[hs_v.at[pl.ds(p * HSS, 2 * H)]], b_, sem).wait()
        pltpu.make_async_copy(
            up_hbm.at[px_v.at[pl.ds(p * PXS + 8, 2)]], u_, sem).wait()
        pltpu.make_async_copy(
            pp_hbm.at[px_v.at[pl.ds(p * PXS + 16, 2)]], pp_, sem).wait()
        pltpu.make_async_copy(
            gs_hbm.at[px_v.at[pl.ds(p * PXS + 16, 2)]], s_, sem).wait()

    start(0, bA, uA, pA, sA, semA)
    start(1, bB, uB, pB, sB, semB)

    def compute_row(b_, u_, pp_, s_, p, sub):
        off = sub * H
        r = 2 * p + sub
        pvec = px_v[pl.ds(p * PXS, 16)]
        cu = pvec[sub]
        ct = pvec[2 + sub]
        hj0 = s_[sub, pl.ds(ct, 16)]
        hj1 = s_[sub, pl.ds(ct + 16, 16)]
        rb = r * DP
        f = [dpf_v[pl.ds(rb + 16 * t, 16)] for t in range(4)]
        for d in range(D):
            hv = hj0 if d < 16 else hj1
            hjd = jnp.broadcast_to(hv[d % 16], (16,))
            for t in range(4):
                w_v[pl.ds(50 * d + 16 * t, 16)] = hjd * f[t]

        u0 = u_[sub, pl.ds(cu, 16)]
        u1 = u_[sub, pl.ds(cu + 16, 16)]
        p0 = pp_[sub, pl.ds(ct, 16)]
        p1 = pp_[sub, pl.ds(ct + 16, 16)]
        acc_tz = u0 * p0 + u1 * p1

        accy = jnp.zeros((16,), jnp.float32)
        for ci in range(4):
            e0 = 16 * ci if ci < 3 else H - 16
            lanes = range(16) if ci < 3 else range(14, 16)
            wbase = (off + e0) // 2
            cvec = hc_v[pl.ds(p * HCS + wbase, 16)]
            for i in lanes:
                e = e0 + i
                oe = off + e
                wi = oe // 2 - wbase
                cb = (cvec[wi] >> (16 * (oe % 2))) & 0xFFFF
                g0 = b_[oe, pl.ds(cb, 16)]
                w0 = w_v[pl.ds(e * 32, 16)]
                g1 = b_[oe, pl.ds(cb + 16, 16)]
                w1 = w_v[pl.ds(e * 32 + 16, 16)]
                accy = accy + g0 * w0 + g1 * w1
        out_v[pl.ds(r * 16, 16)] = acc_tz + accy * (1.0 / H)

    def body(j, c):
        p = 2 * j
        wait(p, bA, uA, pA, sA, semA)
        compute_row(bA, uA, pA, sA, p, 0)
        compute_row(bA, uA, pA, sA, p, 1)

        @pl.when(j < NPAIR // 2 - 1)
        def _():
            start(p + 2, bA, uA, pA, sA, semA)

        wait(p + 1, bB, uB, pB, sB, semB)
        compute_row(bB, uB, pB, sB, p + 1, 0)
        compute_row(bB, uB, pB, sB, p + 1, 1)

        @pl.when(j < NPAIR // 2 - 1)
        def _():
            start(p + 3, bB, uB, pB, sB, semB)
        return c

    lax.fori_loop(0, NPAIR // 2, body, 0)
    pltpu.sync_copy(out_v, out_hbm.at[pl.ds(base * 16, CB * 16)])


def _fin_body(part_ref, cuj_ref, out_s_ref, out_w_ref):
    suj = jnp.sum(part_ref[...], axis=1, keepdims=True)
    out_s_ref[...] = 1.0 / (1.0 + jnp.exp(-suj))
    out_w_ref[...] = 1.0 + jnp.log(1.0 + cuj_ref[...] * (10.0 ** 10))


def kernel(user_id, targets, history, check_in_num, distances,
           UserPreference, PoiPreference, GeoInfluence, GeoSusceptibility):
    ui = user_id.astype(jnp.int32)
    tg = targets.astype(jnp.int32)
    h32 = history.astype(jnp.int32)
    hp = h32.reshape(B // 2, 2 * H)
    hs = jnp.pad(hp >> 2, ((0, 0), (0, HSS - 2 * H))).reshape(-1)
    hcl = (hp & 3) * D
    hc16 = hcl[:, 0::2] | (hcl[:, 1::2] << 16)
    hc = jnp.pad(hc16, ((0, 0), (0, HCS - H))).reshape(-1)
    cu = ((ui & 3) * D).reshape(B // 2, 2)
    ct = ((tg & 3) * D).reshape(B // 2, 2)
    uidx = (ui >> 2).reshape(B // 2, 2)
    tidx = (tg >> 2).reshape(B // 2, 2)
    z4 = jnp.zeros((B // 2, 4), jnp.int32)
    z6 = jnp.zeros((B // 2, 6), jnp.int32)
    z14 = jnp.zeros((B // 2, 14), jnp.int32)
    px = jnp.concatenate(
        [cu, ct, z4, uidx, z6, tidx, z14], axis=1).reshape(-1)
    dist_flat = jnp.pad(distances, ((0, 0), (0, DP - H))).reshape(B * DP)
    up4 = UserPreference.reshape(TR, 128)
    pp4 = PoiPreference.reshape(TR, 128)
    gi4 = GeoInfluence.reshape(TR, 128)
    gs4 = GeoSusceptibility.reshape(TR, 128)

    mesh = plsc.VectorSubcoreMesh(core_axis_name="c", subcore_axis_name="s")
    sc = pl.kernel(
        _sc_body,
        mesh=mesh,
        out_type=jax.ShapeDtypeStruct((B * 16,), jnp.float32),
        scratch_types=[
            pltpu.VMEM((NPAIR * HSS,), jnp.int32),
            pltpu.VMEM((NPAIR * HCS,), jnp.int32),
            pltpu.VMEM((NPAIR * PXS,), jnp.int32),
            pltpu.VMEM((CB * DP,), jnp.float32),
            pltpu.VMEM((2 * H, 128), jnp.float32),
            pltpu.VMEM((2 * H, 128), jnp.float32),
            pltpu.VMEM((2, 128), jnp.float32),
            pltpu.VMEM((2, 128), jnp.float32),
            pltpu.VMEM((2, 128), jnp.float32),
            pltpu.VMEM((2, 128), jnp.float32),
            pltpu.VMEM((2, 128), jnp.float32),
            pltpu.VMEM((2, 128), jnp.float32),
            pltpu.VMEM((1664,), jnp.float32),
            pltpu.VMEM((CB * 16,), jnp.float32),
            pltpu.SemaphoreType.DMA,
            pltpu.SemaphoreType.DMA,
        ],
    )
    part = sc(hs, hc, px, dist_flat, up4, pp4, gi4, gs4).reshape(B, 16)

    out_s, wuj = pl.pallas_call(
        _fin_body,
        grid=(8,),
        in_specs=[
            pl.BlockSpec((B // 8, 16), lambda i: (i, 0)),
            pl.BlockSpec((B // 8, 1), lambda i: (i, 0)),
        ],
        out_specs=[
            pl.BlockSpec((B // 8, 1), lambda i: (i, 0)),
            pl.BlockSpec((B // 8, 1), lambda i: (i, 0)),
        ],
        out_shape=[
            jax.ShapeDtypeStruct((B, 1), jnp.float32),
            jax.ShapeDtypeStruct((B, 1), jnp.float32),
        ],
    )(part, check_in_num)

    return out_s, wuj


# speedup vs baseline: 2.4525x; 2.4525x over previous
import functools

import jax
import jax.numpy as jnp
from jax import lax
from jax.experimental import pallas as pl
from jax.experimental.pallas import tpu as pltpu
from jax.experimental.pallas import tpu_sc as plsc

B = 16384
H = 50
D = 32
DP = 64
NW = 32
CB = B // NW
NPAIR = CB // 2


def _sqrt16(x):
    xs = jnp.maximum(x, 1e-12)
    i = lax.bitcast_convert_type(xs, jnp.int32)
    y = lax.bitcast_convert_type(jnp.int32(0x5F3759DF) - (i >> 1), jnp.float32)
    y = y * (1.5 - 0.5 * xs * y * y)
    y = y * (1.5 - 0.5 * xs * y * y)
    return xs * y


def _sc_body(hist_hbm, dist_hbm, hj_hbm, gi_hbm, out_hbm,
             hist_v, dpf_v, hj_v, gA, gB, w_v, out_v, semA, semB):
    wid = lax.axis_index("c") * 16 + lax.axis_index("s")
    base = wid * CB

    pltpu.sync_copy(hist_hbm.at[pl.ds(wid * NPAIR, NPAIR)], hist_v)
    pltpu.sync_copy(dist_hbm.at[pl.ds(base * DP, CB * DP)], dpf_v)
    pltpu.sync_copy(hj_hbm.at[pl.ds(base, CB)], hj_v)

    def _sqrt_step(i, c):
        sl = pl.ds(i * 16, 16)
        dpf_v[sl] = _sqrt16(dpf_v[sl])
        return c
    lax.fori_loop(0, CB * DP // 16, _sqrt_step, 0)

    def start(p, buf, sem):
        pltpu.async_copy(gi_hbm.at[hist_v.at[p]], buf, sem)

    def wait(p, buf, sem):
        pltpu.make_async_copy(gi_hbm.at[hist_v.at[p]], buf, sem).wait()

    start(0, gA, semA)
    start(1, gB, semB)

    def compute_row(buf, r, off):
        hj0 = hj_v[r, pl.ds(0, 16)]
        hj1 = hj_v[r, pl.ds(16, 16)]
        rb = r * DP
        f = [dpf_v[pl.ds(rb + 16 * t, 16)] for t in range(4)]
        for d in range(D):
            hv = hj0 if d < 16 else hj1
            hjd = jnp.broadcast_to(hv[d % 16], (16,))
            for t in range(4):
                w_v[pl.ds(50 * d + 16 * t, 16)] = hjd * f[t]

        def e_step(e, accy):
            er = off + e
            k0 = e * 32
            g0 = buf[er, pl.ds(0, 16)]
            w0 = w_v[pl.ds(k0, 16)]
            g1 = buf[er, pl.ds(16, 16)]
            w1 = w_v[pl.ds(k0 + 16, 16)]
            return accy + g0 * w0 + g1 * w1

        accy = lax.fori_loop(0, H, e_step, jnp.zeros((16,), jnp.float32))
        out_v[r, pl.ds(0, 16)] = accy * (1.0 / H)

    def body(j, c):
        p = 2 * j
        wait(p, gA, semA)
        compute_row(gA, 2 * p, 0)
        compute_row(gA, 2 * p + 1, H)

        @pl.when(j < NPAIR // 2 - 1)
        def _():
            start(p + 2, gA, semA)

        wait(p + 1, gB, semB)
        compute_row(gB, 2 * p + 2, 0)
        compute_row(gB, 2 * p + 3, H)

        @pl.when(j < NPAIR // 2 - 1)
        def _():
            start(p + 3, gB, semB)
        return c

    lax.fori_loop(0, NPAIR // 2, body, 0)
    pltpu.sync_copy(out_v, out_hbm.at[pl.ds(base, CB)])


def _fin_body(part_ref, up_ref, pp_ref, cuj_ref, out_s_ref, out_w_ref):
    yij = jnp.sum(part_ref[...], axis=1, keepdims=True)
    tz = jnp.sum(up_ref[...] * pp_ref[...], axis=1, keepdims=True)
    suj = tz + yij
    out_s_ref[...] = 1.0 / (1.0 + jnp.exp(-suj))
    out_w_ref[...] = 1.0 + jnp.log(1.0 + cuj_ref[...] * (10.0 ** 10))


def kernel(user_id, targets, history, check_in_num, distances,
           UserPreference, PoiPreference, GeoInfluence, GeoSusceptibility):
    hist2 = history.astype(jnp.int32).reshape(B // 2, 2 * H)
    dist_flat = jnp.pad(distances, ((0, 0), (0, DP - H))).reshape(B * DP)
    hj = jnp.take(GeoSusceptibility, targets, axis=0)
    up = jnp.take(UserPreference, user_id, axis=0)
    pp = jnp.take(PoiPreference, targets, axis=0)

    mesh = plsc.VectorSubcoreMesh(core_axis_name="c", subcore_axis_name="s")
    sc = pl.kernel(
        _sc_body,
        mesh=mesh,
        compiler_params=pltpu.CompilerParams(use_tc_tiling_on_sc=False),
        out_type=jax.ShapeDtypeStruct((B, 16), jnp.float32),
        scratch_types=[
            pltpu.VMEM((NPAIR, 2 * H), jnp.int32),
            pltpu.VMEM((CB * DP,), jnp.float32),
            pltpu.VMEM((CB, D), jnp.float32),
            pltpu.VMEM((2 * H, D), jnp.float32),
            pltpu.VMEM((2 * H, D), jnp.float32),
            pltpu.VMEM((1664,), jnp.float32),
            pltpu.VMEM((CB, 16), jnp.float32),
            pltpu.SemaphoreType.DMA,
            pltpu.SemaphoreType.DMA,
        ],
    )
    part = sc(hist2, dist_flat, hj, GeoInfluence)

    out_s, wuj = pl.pallas_call(
        _fin_body,
        grid=(8,),
        in_specs=[
            pl.BlockSpec((B // 8, 16), lambda i: (i, 0)),
            pl.BlockSpec((B // 8, D), lambda i: (i, 0)),
            pl.BlockSpec((B // 8, D), lambda i: (i, 0)),
            pl.BlockSpec((B // 8, 1), lambda i: (i, 0)),
        ],
        out_specs=[
            pl.BlockSpec((B // 8, 1), lambda i: (i, 0)),
            pl.BlockSpec((B // 8, 1), lambda i: (i, 0)),
        ],
        out_shape=[
            jax.ShapeDtypeStruct((B, 1), jnp.float32),
            jax.ShapeDtypeStruct((B, 1), jnp.float32),
        ],
    )(part, up, pp, check_in_num)

    return out_s, wuj
